# baseline (speedup 1.0000x reference)
import functools

import jax
import jax.numpy as jnp
from jax import lax
from jax.experimental import pallas as pl
from jax.experimental.pallas import tpu as pltpu
from jax.experimental.pallas import tpu_sc as plsc

_N = 10000
_E = 320000
_IN_DIM = 128
_HID = 64

_NC = 2
_NS = 16
_NW = _NC * _NS
_EB = 4096
_GPB = _EB // 16

_E_PAD = 327680
_EXTRA = 2 * _EB

_N_ACC = 10240

_NB = 10
_BLK = 1000

_SC_F = 4
_SC_EPR = _E_PAD // _NC
_SC_NBLK = _SC_EPR // _EB
_DT_F = 8
_DT_NR = 4
_DT_EPR = _E_PAD // _DT_NR
_DT_NBLK = _DT_EPR // _EB

_MESH = plsc.VectorSubcoreMesh(core_axis_name="c", subcore_axis_name="s")


@functools.partial(
    pl.kernel,
    out_type=jax.ShapeDtypeStruct((_NC, _NB, _HID, _BLK), jnp.float32),
    mesh=_MESH,
    scratch_types=[
        pltpu.VMEM((_SC_F, _N), jnp.float32),
        pltpu.VMEM((_SC_F, _N_ACC), jnp.float32),
        pltpu.VMEM((2, _EB), jnp.int32),
        pltpu.VMEM((2, _EB), jnp.int32),
        pltpu.SemaphoreType.DMA,
        pltpu.SemaphoreType.DMA,
        pltpu.SemaphoreType.DMA,
        pltpu.SemaphoreType.DMA,
        pltpu.SemaphoreType.DMA,
    ],
    compiler_params=pltpu.CompilerParams(use_tc_tiling_on_sc=False,
                                         needs_layout_passes=False),
)
def _sc_scatter_add(ht_hbm, src_hbm, dst_hbm, out_hbm,
                    tbl_v, acc_v, src_v, dst_v,
                    sem_t, sem_s0, sem_d0, sem_s1, sem_d1):
    c = lax.axis_index("c")
    s = lax.axis_index("s")
    base = c * _SC_EPR
    ssems = (sem_s0, sem_s1)
    dsems = (sem_d0, sem_d1)

    cts = [pltpu.async_copy(ht_hbm.at[nb, pl.ds(s * _SC_F, _SC_F)],
                            tbl_v.at[:, pl.ds(nb * _BLK, _BLK)],
                            sem_t)
           for nb in range(_NB)]
    pltpu.async_copy(src_hbm.at[pl.ds(base, _EB)], src_v.at[0], sem_s0)
    pltpu.async_copy(dst_hbm.at[pl.ds(base, _EB)], dst_v.at[0], sem_d0)
    pltpu.async_copy(src_hbm.at[pl.ds(base + _EB, _EB)], src_v.at[1], sem_s1)
    pltpu.async_copy(dst_hbm.at[pl.ds(base + _EB, _EB)], dst_v.at[1], sem_d1)

    for f in range(_SC_F):
        @plsc.parallel_loop(0, _N_ACC // 16, unroll=8)
        def _zero(i, f=f):
            acc_v[f, pl.ds(i * 16, 16)] = jnp.zeros((16,), jnp.float32)

    for ct in cts:
        ct.wait()

    fsplats = [jnp.full((16,), f, jnp.int32) for f in range(_SC_F)]

    def do_block(b, slot):
        sref = src_v.at[slot]
        dref = dst_v.at[slot]
        pltpu.make_async_copy(src_hbm.at[pl.ds(0, _EB)], sref,
                              ssems[slot]).wait()
        pltpu.make_async_copy(dst_hbm.at[pl.ds(0, _EB)], dref,
                              dsems[slot]).wait()

        @plsc.parallel_loop(0, _GPB, unroll=8)
        def _grp(g):
            src16 = sref[pl.ds(g * 16, 16)]
            dst16 = dref[pl.ds(g * 16, 16)]
            for f in range(_SC_F):
                v = plsc.load_gather(tbl_v, [fsplats[f], src16])
                plsc.addupdate_scatter(acc_v, [fsplats[f], dst16], v)

        off = base + (b + 2) * _EB
        pltpu.async_copy(src_hbm.at[pl.ds(off, _EB)], sref, ssems[slot])
        pltpu.async_copy(dst_hbm.at[pl.ds(off, _EB)], dref, dsems[slot])

    def body(bb, carry):
        do_block(bb * 2, 0)
        do_block(bb * 2 + 1, 1)
        return carry

    lax.fori_loop(0, _SC_NBLK // 2, body, 0)
    for sem in (sem_s0, sem_d0, sem_s1, sem_d1):
        pltpu.make_async_copy(src_hbm.at[pl.ds(0, _EB)], src_v.at[0],
                              sem).wait()
    for nb in range(_NB):
        pltpu.sync_copy(acc_v.at[:, pl.ds(nb * _BLK, _BLK)],
                        out_hbm.at[c, nb, pl.ds(s * _SC_F, _SC_F)])


@functools.partial(
    pl.kernel,
    out_type=jax.ShapeDtypeStruct((_DT_F, _E_PAD), jnp.float32),
    mesh=_MESH,
    scratch_types=[
        pltpu.VMEM((_DT_F, _N), jnp.float32),
        pltpu.VMEM((2, _EB), jnp.int32),
        pltpu.VMEM((2, _EB), jnp.int32),
        pltpu.VMEM((2, _EB), jnp.float32),
        pltpu.SemaphoreType.DMA,
        pltpu.SemaphoreType.DMA,
        pltpu.SemaphoreType.DMA,
        pltpu.SemaphoreType.DMA,
        pltpu.SemaphoreType.DMA,
        pltpu.SemaphoreType.DMA,
        pltpu.SemaphoreType.DMA,
    ],
    compiler_params=pltpu.CompilerParams(use_tc_tiling_on_sc=False,
                                         needs_layout_passes=False),
)
def _sc_edge_dot(ht_hbm, src_hbm, dst_hbm, out_hbm,
                 tbl_v, src_v, dst_v, o_v,
                 sem_t, sem_s0, sem_d0, sem_s1, sem_d1, sem_o0, sem_o1):
    c = lax.axis_index("c")
    s = lax.axis_index("s")
    k = lax.rem(s, _DT_F)
    r = lax.div(s, _DT_F) * _NC + c
    base = r * _DT_EPR
    ssems = (sem_s0, sem_s1)
    dsems = (sem_d0, sem_d1)
    osems = (sem_o0, sem_o1)

    cts = [pltpu.async_copy(ht_hbm.at[nb, pl.ds(k * _DT_F, _DT_F)],
                            tbl_v.at[:, pl.ds(nb * _BLK, _BLK)],
                            sem_t)
           for nb in range(_NB)]
    pltpu.async_copy(src_hbm.at[pl.ds(base, _EB)], src_v.at[0], sem_s0)
    pltpu.async_copy(dst_hbm.at[pl.ds(base, _EB)], dst_v.at[0], sem_d0)
    pltpu.async_copy(src_hbm.at[pl.ds(base + _EB, _EB)], src_v.at[1], sem_s1)
    pltpu.async_copy(dst_hbm.at[pl.ds(base + _EB, _EB)], dst_v.at[1], sem_d1)
    for ct in cts:
        ct.wait()

    fsplats = [jnp.full((16,), f, jnp.int32) for f in range(_DT_F)]

    def do_block(b, slot, first):
        sref = src_v.at[slot]
        dref = dst_v.at[slot]
        oref = o_v.at[slot]
        pltpu.make_async_copy(src_hbm.at[pl.ds(0, _EB)], sref,
                              ssems[slot]).wait()
        pltpu.make_async_copy(dst_hbm.at[pl.ds(0, _EB)], dref,
                              dsems[slot]).wait()

        @pl.when(jnp.logical_not(first))
        def _():
            pltpu.make_async_copy(oref, out_hbm.at[k, pl.ds(0, _EB)],
                                  osems[slot]).wait()

        @plsc.parallel_loop(0, _GPB, unroll=4)
        def _grp(g):
            src16 = sref[pl.ds(g * 16, 16)]
            dst16 = dref[pl.ds(g * 16, 16)]
            acc0 = jnp.zeros((16,), jnp.float32)
            acc1 = jnp.zeros((16,), jnp.float32)
            for f in range(_DT_F):
                va = plsc.load_gather(tbl_v, [fsplats[f], src16])
                vb = plsc.load_gather(tbl_v, [fsplats[f], dst16])
                if f % 2 == 0:
                    acc0 = acc0 + va * vb
                else:
                    acc1 = acc1 + va * vb
            o_v[slot, pl.ds(g * 16, 16)] = acc0 + acc1

        pltpu.async_copy(oref, out_hbm.at[k, pl.ds(base + b * _EB, _EB)],
                         osems[slot])
        off = base + (b + 2) * _EB
        pltpu.async_copy(src_hbm.at[pl.ds(off, _EB)], sref, ssems[slot])
        pltpu.async_copy(dst_hbm.at[pl.ds(off, _EB)], dref, dsems[slot])

    def body(bb, carry):
        do_block(bb * 2, 0, bb == 0)
        do_block(bb * 2 + 1, 1, bb == 0)
        return carry

    lax.fori_loop(0, _DT_NBLK // 2, body, 0)
    for sem in (sem_s0, sem_d0, sem_s1, sem_d1):
        pltpu.make_async_copy(src_hbm.at[pl.ds(0, _EB)], src_v.at[0],
                              sem).wait()
    for slot in (0, 1):
        pltpu.make_async_copy(o_v.at[slot], out_hbm.at[0, pl.ds(0, _EB)],
                              osems[slot]).wait()


def _matmul(a, w):
    return lax.dot_general(a, w, (((1,), (0,)), ((), ())),
                           preferred_element_type=jnp.float32)


_T_SHAPE = jax.ShapeDtypeStruct((_NB, _HID, _BLK), jnp.float32)
_T_SPEC = pl.BlockSpec((1, _HID, _BLK), lambda i: (i, 0, 0))


def _tc_linear(x, w, b, relu, mode):
    n, kdim = x.shape
    m = w.shape[1]

    def body(x_ref, w_ref, b_ref, *o_refs):
        y = _matmul(x_ref[...], w_ref[...]) + b_ref[...]
        if relu:
            y = jnp.maximum(y, 0.0)
        if mode == "plain_t":
            o_refs[0][...] = y
            o_refs[1][0] = y.T
        else:
            o_refs[0][0] = y.T

    if mode == "plain_t":
        out_shape = [jax.ShapeDtypeStruct((n, m), jnp.float32), _T_SHAPE]
        out_specs = [pl.BlockSpec((_BLK, m), lambda i: (i, 0)), _T_SPEC]
    else:
        out_shape = [_T_SHAPE]
        out_specs = [_T_SPEC]

    return pl.pallas_call(
        body,
        grid=(n // _BLK,),
        in_specs=[
            pl.BlockSpec((_BLK, kdim), lambda i: (i, 0)),
            pl.BlockSpec((kdim, m), lambda i: (0, 0)),
            pl.BlockSpec((1, m), lambda i: (0, 0)),
        ],
        out_specs=out_specs,
        out_shape=out_shape,
    )(x, w, b.reshape(1, m))


def _tc_gin_mlp(h, agg, w1, b1, w2, b2, relu_out, emit_t):
    n = h.shape[0]
    m = w2.shape[1]

    def body(h_ref, a_ref, w1_ref, b1_ref, w2_ref, b2_ref, *o_refs):
        asum = a_ref[0, 0] + a_ref[1, 0]
        z = h_ref[...] + asum.T
        t = jnp.maximum(_matmul(z, w1_ref[...]) + b1_ref[...], 0.0)
        y = _matmul(t, w2_ref[...]) + b2_ref[...]
        if relu_out:
            y = jnp.maximum(y, 0.0)
        o_refs[0][...] = y
        if emit_t:
            o_refs[1][0] = y.T

    out_shape = [jax.ShapeDtypeStruct((n, m), jnp.float32)]
    out_specs = [pl.BlockSpec((_BLK, m), lambda i: (i, 0))]
    if emit_t:
        out_shape.append(_T_SHAPE)
        out_specs.append(_T_SPEC)

    return pl.pallas_call(
        body,
        grid=(n // _BLK,),
        in_specs=[
            pl.BlockSpec((_BLK, _HID), lambda i: (i, 0)),
            pl.BlockSpec((_NC, 1, _HID, _BLK), lambda i: (0, i, 0, 0)),
            pl.BlockSpec((_HID, _HID), lambda i: (0, 0)),
            pl.BlockSpec((1, _HID), lambda i: (0, 0)),
            pl.BlockSpec((_HID, m), lambda i: (0, 0)),
            pl.BlockSpec((1, m), lambda i: (0, 0)),
        ],
        out_specs=out_specs,
        out_shape=out_shape,
    )(h, agg, w1, b1.reshape(1, _HID), w2, b2.reshape(1, m))


def _tc_slice_sum(parts):
    blk = 32768

    def body(p_ref, o_ref):
        o_ref[...] = jnp.sum(p_ref[...], axis=0)

    return pl.pallas_call(
        body,
        grid=(_E_PAD // blk,),
        in_specs=[pl.BlockSpec((_DT_F, blk), lambda i: (0, i))],
        out_specs=pl.BlockSpec((blk,), lambda i: (i,)),
        out_shape=jax.ShapeDtypeStruct((_E_PAD,), jnp.float32),
    )(parts)


def kernel(x, edge_index, W_lin, b_lin, enc1_W1, enc1_b1, enc1_W2, enc1_b2,
           enc2_W1, enc2_b1, enc2_W2, enc2_b2, dec1_W1, dec1_b1, dec1_W2,
           dec1_b2, dec2_W1, dec2_b1, dec2_W2, dec2_b2, Ws, bs):
    src = edge_index[0]
    dst = edge_index[1]
    pad = _E_PAD - _E
    srcp = jnp.concatenate(
        [src, jnp.zeros((pad + _EXTRA,), jnp.int32)])
    pad_rows = _N + (jnp.arange(pad + _EXTRA, dtype=jnp.int32)
                     % (_N_ACC - _N))
    dst_sc = jnp.concatenate([dst, pad_rows])
    dst_g = jnp.concatenate([dst, jnp.zeros((pad + _EXTRA,), jnp.int32)])

    def gin(h, ht, w1, b1, w2, b2, relu_out, emit_t):
        agg = _sc_scatter_add(ht, srcp, dst_sc)
        return _tc_gin_mlp(h, agg, w1, b1, w2, b2, relu_out, emit_t)

    h0, h0t = _tc_linear(x, W_lin, b_lin, False, "plain_t")
    h1, h1t = gin(h0, h0t, enc1_W1, enc1_b1, enc1_W2, enc1_b2, True, True)
    emb, embt = gin(h1, h1t, enc2_W1, enc2_b1, enc2_W2, enc2_b2, False, True)
    a, at_ = gin(emb, embt, dec1_W1, dec1_b1, dec1_W2, dec1_b2, True, True)
    (x_,) = gin(a, at_, dec2_W1, dec2_b1, dec2_W2, dec2_b2, False, False)

    (hst,) = _tc_linear(emb, Ws, bs, True, "t_only")
    parts = _sc_edge_dot(hst, srcp, dst_g)
    s_ = _tc_slice_sum(parts)[:_E]
    return (x_, s_)


# speedup vs baseline: 1.0275x; 1.0275x over previous
import functools

import jax
import jax.numpy as jnp
from jax import lax
from jax.experimental import pallas as pl
from jax.experimental.pallas import tpu as pltpu
from jax.experimental.pallas import tpu_sc as plsc

_N = 10000
_E = 320000
_IN_DIM = 128
_HID = 64

_NC = 2
_NS = 16
_NW = _NC * _NS
_EB = 4096
_GPB = _EB // 16

_E_PAD = 327680
_EXTRA = 2 * _EB

_N_ACC = 10240

_NB = 10
_BLK = 1000

_SC_F = 4
_SC_EPR = _E_PAD // _NC
_SC_NBLK = _SC_EPR // _EB
_DT_F = 8
_DT_NR = 4
_DT_EPR = _E_PAD // _DT_NR
_DT_NBLK = _DT_EPR // _EB

_MESH = plsc.VectorSubcoreMesh(core_axis_name="c", subcore_axis_name="s")


@functools.partial(
    pl.kernel,
    out_type=jax.ShapeDtypeStruct((_NC, _NB, _HID, _BLK), jnp.float32),
    mesh=_MESH,
    scratch_types=[
        pltpu.VMEM((_SC_F, _N), jnp.float32),
        pltpu.VMEM((_SC_F, _N_ACC), jnp.float32),
        pltpu.VMEM((2, _EB), jnp.int32),
        pltpu.SemaphoreType.DMA,
        pltpu.SemaphoreType.DMA,
        pltpu.SemaphoreType.DMA,
    ],
    compiler_params=pltpu.CompilerParams(use_tc_tiling_on_sc=False,
                                         needs_layout_passes=False),
)
def _sc_scatter_add(ht_hbm, edge_hbm, out_hbm,
                    tbl_v, acc_v, e_v,
                    sem_t, sem_e0, sem_e1):
    c = lax.axis_index("c")
    s = lax.axis_index("s")
    base = c * _SC_EPR
    esems = (sem_e0, sem_e1)

    cts = [pltpu.async_copy(ht_hbm.at[nb, pl.ds(s * _SC_F, _SC_F)],
                            tbl_v.at[:, pl.ds(nb * _BLK, _BLK)],
                            sem_t)
           for nb in range(_NB)]
    pltpu.async_copy(edge_hbm.at[pl.ds(base, _EB)], e_v.at[0], sem_e0)
    pltpu.async_copy(edge_hbm.at[pl.ds(base + _EB, _EB)], e_v.at[1], sem_e1)

    for f in range(_SC_F):
        @plsc.parallel_loop(0, _N_ACC // 16, unroll=8)
        def _zero(i, f=f):
            acc_v[f, pl.ds(i * 16, 16)] = jnp.zeros((16,), jnp.float32)

    for ct in cts:
        ct.wait()

    fsplats = [jnp.full((16,), f, jnp.int32) for f in range(_SC_F)]
    lomask = jnp.full((16,), 0xFFFF, jnp.int32)

    def do_block(b, slot):
        eref = e_v.at[slot]
        pltpu.make_async_copy(edge_hbm.at[pl.ds(0, _EB)], eref,
                              esems[slot]).wait()

        @plsc.parallel_loop(0, _GPB, unroll=4)
        def _grp(g):
            e16 = eref[pl.ds(g * 16, 16)]
            src16 = jnp.bitwise_and(e16, lomask)
            dst16 = lax.shift_right_logical(e16, 16)
            for f in range(_SC_F):
                v = plsc.load_gather(tbl_v, [fsplats[f], src16])
                plsc.addupdate_scatter(acc_v, [fsplats[f], dst16], v)

        off = base + (b + 2) * _EB
        pltpu.async_copy(edge_hbm.at[pl.ds(off, _EB)], eref, esems[slot])

    def body(bb, carry):
        do_block(bb * 2, 0)
        do_block(bb * 2 + 1, 1)
        return carry

    lax.fori_loop(0, _SC_NBLK // 2, body, 0)
    for sem in (sem_e0, sem_e1):
        pltpu.make_async_copy(edge_hbm.at[pl.ds(0, _EB)], e_v.at[0],
                              sem).wait()
    for nb in range(_NB):
        pltpu.sync_copy(acc_v.at[:, pl.ds(nb * _BLK, _BLK)],
                        out_hbm.at[c, nb, pl.ds(s * _SC_F, _SC_F)])


@functools.partial(
    pl.kernel,
    out_type=jax.ShapeDtypeStruct((_DT_F, _E_PAD), jnp.float32),
    mesh=_MESH,
    scratch_types=[
        pltpu.VMEM((_DT_F, _N), jnp.float32),
        pltpu.VMEM((2, _EB), jnp.int32),
        pltpu.VMEM((2, _EB), jnp.float32),
        pltpu.SemaphoreType.DMA,
        pltpu.SemaphoreType.DMA,
        pltpu.SemaphoreType.DMA,
        pltpu.SemaphoreType.DMA,
        pltpu.SemaphoreType.DMA,
    ],
    compiler_params=pltpu.CompilerParams(use_tc_tiling_on_sc=False,
                                         needs_layout_passes=False),
)
def _sc_edge_dot(ht_hbm, edge_hbm, out_hbm,
                 tbl_v, e_v, o_v,
                 sem_t, sem_e0, sem_e1, sem_o0, sem_o1):
    c = lax.axis_index("c")
    s = lax.axis_index("s")
    k = lax.rem(s, _DT_F)
    r = lax.div(s, _DT_F) * _NC + c
    base = r * _DT_EPR
    esems = (sem_e0, sem_e1)
    osems = (sem_o0, sem_o1)

    cts = [pltpu.async_copy(ht_hbm.at[nb, pl.ds(k * _DT_F, _DT_F)],
                            tbl_v.at[:, pl.ds(nb * _BLK, _BLK)],
                            sem_t)
           for nb in range(_NB)]
    pltpu.async_copy(edge_hbm.at[pl.ds(base, _EB)], e_v.at[0], sem_e0)
    pltpu.async_copy(edge_hbm.at[pl.ds(base + _EB, _EB)], e_v.at[1], sem_e1)
    for ct in cts:
        ct.wait()

    fsplats = [jnp.full((16,), f, jnp.int32) for f in range(_DT_F)]
    lomask = jnp.full((16,), 0xFFFF, jnp.int32)

    def do_block(b, slot, first):
        eref = e_v.at[slot]
        oref = o_v.at[slot]
        pltpu.make_async_copy(edge_hbm.at[pl.ds(0, _EB)], eref,
                              esems[slot]).wait()

        @pl.when(jnp.logical_not(first))
        def _():
            pltpu.make_async_copy(oref, out_hbm.at[k, pl.ds(0, _EB)],
                                  osems[slot]).wait()

        @plsc.parallel_loop(0, _GPB, unroll=4)
        def _grp(g):
            e16 = eref[pl.ds(g * 16, 16)]
            src16 = jnp.bitwise_and(e16, lomask)
            dst16 = lax.shift_right_logical(e16, 16)
            acc0 = jnp.zeros((16,), jnp.float32)
            acc1 = jnp.zeros((16,), jnp.float32)
            for f in range(_DT_F):
                va = plsc.load_gather(tbl_v, [fsplats[f], src16])
                vb = plsc.load_gather(tbl_v, [fsplats[f], dst16])
                if f % 2 == 0:
                    acc0 = acc0 + va * vb
                else:
                    acc1 = acc1 + va * vb
            o_v[slot, pl.ds(g * 16, 16)] = acc0 + acc1

        pltpu.async_copy(oref, out_hbm.at[k, pl.ds(base + b * _EB, _EB)],
                         osems[slot])
        off = base + (b + 2) * _EB
        pltpu.async_copy(edge_hbm.at[pl.ds(off, _EB)], eref, esems[slot])

    def body(bb, carry):
        do_block(bb * 2, 0, bb == 0)
        do_block(bb * 2 + 1, 1, bb == 0)
        return carry

    lax.fori_loop(0, _DT_NBLK // 2, body, 0)
    for sem in (sem_e0, sem_e1):
        pltpu.make_async_copy(edge_hbm.at[pl.ds(0, _EB)], e_v.at[0],
                              sem).wait()
    for slot in (0, 1):
        pltpu.make_async_copy(o_v.at[slot], out_hbm.at[0, pl.ds(0, _EB)],
                              osems[slot]).wait()


def _matmul(a, w):
    return lax.dot_general(a, w, (((1,), (0,)), ((), ())),
                           preferred_element_type=jnp.float32)


_T_SHAPE = jax.ShapeDtypeStruct((_NB, _HID, _BLK), jnp.float32)
_T_SPEC = pl.BlockSpec((1, _HID, _BLK), lambda i: (i, 0, 0))


def _tc_linear(x, w, b, relu, mode):
    n, kdim = x.shape
    m = w.shape[1]

    def body(x_ref, w_ref, b_ref, *o_refs):
        y = _matmul(x_ref[...], w_ref[...]) + b_ref[...]
        if relu:
            y = jnp.maximum(y, 0.0)
        if mode == "plain_t":
            o_refs[0][...] = y
            o_refs[1][0] = y.T
        else:
            o_refs[0][0] = y.T

    if mode == "plain_t":
        out_shape = [jax.ShapeDtypeStruct((n, m), jnp.float32), _T_SHAPE]
        out_specs = [pl.BlockSpec((_BLK, m), lambda i: (i, 0)), _T_SPEC]
    else:
        out_shape = [_T_SHAPE]
        out_specs = [_T_SPEC]

    return pl.pallas_call(
        body,
        grid=(n // _BLK,),
        in_specs=[
            pl.BlockSpec((_BLK, kdim), lambda i: (i, 0)),
            pl.BlockSpec((kdim, m), lambda i: (0, 0)),
            pl.BlockSpec((1, m), lambda i: (0, 0)),
        ],
        out_specs=out_specs,
        out_shape=out_shape,
    )(x, w, b.reshape(1, m))


def _tc_gin_mlp(h, agg, w1, b1, w2, b2, relu_out, emit_t):
    n = h.shape[0]
    m = w2.shape[1]

    def body(h_ref, a_ref, w1_ref, b1_ref, w2_ref, b2_ref, *o_refs):
        asum = a_ref[0, 0] + a_ref[1, 0]
        z = h_ref[...] + asum.T
        t = jnp.maximum(_matmul(z, w1_ref[...]) + b1_ref[...], 0.0)
        y = _matmul(t, w2_ref[...]) + b2_ref[...]
        if relu_out:
            y = jnp.maximum(y, 0.0)
        o_refs[0][...] = y
        if emit_t:
            o_refs[1][0] = y.T

    out_shape = [jax.ShapeDtypeStruct((n, m), jnp.float32)]
    out_specs = [pl.BlockSpec((_BLK, m), lambda i: (i, 0))]
    if emit_t:
        out_shape.append(_T_SHAPE)
        out_specs.append(_T_SPEC)

    return pl.pallas_call(
        body,
        grid=(n // _BLK,),
        in_specs=[
            pl.BlockSpec((_BLK, _HID), lambda i: (i, 0)),
            pl.BlockSpec((_NC, 1, _HID, _BLK), lambda i: (0, i, 0, 0)),
            pl.BlockSpec((_HID, _HID), lambda i: (0, 0)),
            pl.BlockSpec((1, _HID), lambda i: (0, 0)),
            pl.BlockSpec((_HID, m), lambda i: (0, 0)),
            pl.BlockSpec((1, m), lambda i: (0, 0)),
        ],
        out_specs=out_specs,
        out_shape=out_shape,
    )(h, agg, w1, b1.reshape(1, _HID), w2, b2.reshape(1, m))


def _tc_slice_sum(parts):
    blk = 32768

    def body(p_ref, o_ref):
        o_ref[...] = jnp.sum(p_ref[...], axis=0)

    return pl.pallas_call(
        body,
        grid=(_E_PAD // blk,),
        in_specs=[pl.BlockSpec((_DT_F, blk), lambda i: (0, i))],
        out_specs=pl.BlockSpec((blk,), lambda i: (i,)),
        out_shape=jax.ShapeDtypeStruct((_E_PAD,), jnp.float32),
    )(parts)


def kernel(x, edge_index, W_lin, b_lin, enc1_W1, enc1_b1, enc1_W2, enc1_b2,
           enc2_W1, enc2_b1, enc2_W2, enc2_b2, dec1_W1, dec1_b1, dec1_W2,
           dec1_b2, dec2_W1, dec2_b1, dec2_W2, dec2_b2, Ws, bs):
    src = edge_index[0]
    dst = edge_index[1]
    pad = _E_PAD - _E
    srcp = jnp.concatenate(
        [src, jnp.zeros((pad + _EXTRA,), jnp.int32)])
    pad_rows = _N + (jnp.arange(pad + _EXTRA, dtype=jnp.int32)
                     % (_N_ACC - _N))
    dst_sc = jnp.concatenate([dst, pad_rows])
    dst_g = jnp.concatenate([dst, jnp.zeros((pad + _EXTRA,), jnp.int32)])
    edges_sc = jnp.bitwise_or(srcp, jnp.left_shift(dst_sc, 16))
    edges_g = jnp.bitwise_or(srcp, jnp.left_shift(dst_g, 16))

    def gin(h, ht, w1, b1, w2, b2, relu_out, emit_t):
        agg = _sc_scatter_add(ht, edges_sc)
        return _tc_gin_mlp(h, agg, w1, b1, w2, b2, relu_out, emit_t)

    h0, h0t = _tc_linear(x, W_lin, b_lin, False, "plain_t")
    h1, h1t = gin(h0, h0t, enc1_W1, enc1_b1, enc1_W2, enc1_b2, True, True)
    emb, embt = gin(h1, h1t, enc2_W1, enc2_b1, enc2_W2, enc2_b2, False, True)
    a, at_ = gin(emb, embt, dec1_W1, dec1_b1, dec1_W2, dec1_b2, True, True)
    (x_,) = gin(a, at_, dec2_W1, dec2_b1, dec2_W2, dec2_b2, False, False)

    (hst,) = _tc_linear(emb, Ws, bs, True, "t_only")
    parts = _sc_edge_dot(hst, edges_g)
    s_ = _tc_slice_sum(parts)[:_E]
    return (x_, s_)
